# baseline (speedup 1.0000x reference)
import jax
import jax.numpy as jnp
from jax import lax
from jax.experimental import pallas as pl
from jax.experimental.pallas import tpu as pltpu
from jax.experimental.pallas import tpu_sc as plsc

N = 1024
D = 128
NC = 2
NS = 16
L = 16
NW = NC * NS
RPW = N // NW
HPW = RPW // 2
CPR = N // L


def _dist_kernel(seq_ref, q_ref, u_ref, w_ref, b_ref, d_ref, scal_ref):
    s = seq_ref[:]
    q = q_ref[:]
    u = u_ref[:]

    a = s * q
    b = s * u

    dn = (((1,), (1,)), ((), ()))
    g = lax.dot_general(a, b, dn, preferred_element_type=jnp.float32)

    ra = jnp.sum(a * a, axis=1, keepdims=True)
    rs = jnp.sum(s * s, axis=1, keepdims=True)

    ones_row = jnp.ones((1, D), dtype=jnp.float32)
    rb_t = lax.dot_general(ones_row, b * b, dn, preferred_element_type=jnp.float32)
    ones_1 = jnp.ones((1, 1), dtype=jnp.float32)
    uu_t = lax.dot_general(ones_1, u * u, dn, preferred_element_type=jnp.float32)

    d2 = jnp.maximum(ra + rb_t - 2.0 * g, 0.0)
    d = jnp.sqrt(d2)

    diag = jnp.abs(q - u) * jnp.sqrt(rs)
    row_i = lax.broadcasted_iota(jnp.int32, (N, N), 0)
    col_i = lax.broadcasted_iota(jnp.int32, (N, N), 1)
    d = jnp.where(row_i == col_i, diag, d)

    norm = jnp.mean(d)
    d_ref[:] = jnp.where(uu_t > 0.0, d, jnp.inf)

    ri = lax.broadcasted_iota(jnp.int32, (8, L), 0)
    scal = jnp.where(ri == 0, norm,
                     jnp.where(ri == 1, w_ref[0, 0],
                               jnp.where(ri == 2, b_ref[0, 0], 0.0)))
    scal_ref[:] = scal.astype(jnp.float32)


def _vgather(x, idx):
    dnums = lax.GatherDimensionNumbers(
        offset_dims=(), collapsed_slice_dims=(0,), start_index_map=(0,))
    return lax.gather(x, idx[:, None], dnums, slice_sizes=(1,),
                      mode=lax.GatherScatterMode.PROMISE_IN_BOUNDS)


def _sc_min_kernel(d_hbm, scal_hbm, out_hbm, dvm_a, dvm_b, scalvm, outvm,
                   sem_a, sem_b):
    cid = lax.axis_index("c")
    sid = lax.axis_index("s")
    wid = sid * NC + cid
    base = wid * RPW

    cp_a = pltpu.async_copy(d_hbm.at[pl.ds(base, HPW), :], dvm_a, sem_a)
    cp_b = pltpu.async_copy(d_hbm.at[pl.ds(base + HPW, HPW), :], dvm_b, sem_b)
    pltpu.sync_copy(scal_hbm.at[pl.ds(0, 3), :], scalvm)

    normv = scalvm[0, pl.ds(0, L)]
    wv = scalvm[1, pl.ds(0, L)]
    bv = scalvm[2, pl.ds(0, L)]

    lane = lax.iota(jnp.int32, L)
    inf_v = jnp.full((L,), jnp.inf, dtype=jnp.float32)
    zero = jnp.zeros((L,), dtype=jnp.float32)
    ov0, ov1 = zero, zero

    def do_row(dvm, r, ov, sel_lane):
        mn0, mn1, mn2, mn3 = inf_v, inf_v, inf_v, inf_v
        for c in range(0, CPR, 4):
            o = c * L
            mn0 = jnp.minimum(mn0, dvm[r, pl.ds(o, L)])
            mn1 = jnp.minimum(mn1, dvm[r, pl.ds(o + L, L)])
            mn2 = jnp.minimum(mn2, dvm[r, pl.ds(o + 2 * L, L)])
            mn3 = jnp.minimum(mn3, dvm[r, pl.ds(o + 3 * L, L)])
        mn = jnp.minimum(jnp.minimum(mn0, mn1), jnp.minimum(mn2, mn3))
        for s in (8, 4, 2, 1):
            mn = jnp.minimum(mn, _vgather(mn, lane ^ s))
        return jnp.where(lane == sel_lane, mn, ov)

    cp_a.wait()
    for r in range(HPW):
        ov0 = do_row(dvm_a, r, ov0, r)
    cp_b.wait()
    for r in range(HPW):
        ov1 = do_row(dvm_b, r, ov1, r)

    ov0 = jnp.minimum(ov0, normv)
    ov1 = jnp.minimum(ov1, normv)
    outvm[pl.ds(0, L)] = (1.0 - ov0 / normv) * wv + bv
    outvm[pl.ds(L, L)] = (1.0 - ov1 / normv) * wv + bv

    pltpu.sync_copy(outvm, out_hbm.at[pl.ds(base, RPW)])


def kernel(seq, qvs_idx, sum_idx, weight, bias):
    d, scal = pl.pallas_call(
        _dist_kernel,
        out_shape=(
            jax.ShapeDtypeStruct((N, N), jnp.float32),
            jax.ShapeDtypeStruct((8, L), jnp.float32),
        ),
    )(seq, qvs_idx, sum_idx, weight, bias.reshape(1, 1))

    mesh = plsc.VectorSubcoreMesh(core_axis_name="c", subcore_axis_name="s")
    out = pl.kernel(
        _sc_min_kernel,
        mesh=mesh,
        out_type=jax.ShapeDtypeStruct((N,), jnp.float32),
        scratch_types=[
            pltpu.VMEM((HPW, N), jnp.float32),
            pltpu.VMEM((HPW, N), jnp.float32),
            pltpu.VMEM((3, L), jnp.float32),
            pltpu.VMEM((RPW,), jnp.float32),
            pltpu.SemaphoreType.DMA,
            pltpu.SemaphoreType.DMA,
        ],
    )(d, scal)

    return out.reshape(N, 1)


# speedup vs baseline: 1.2965x; 1.2965x over previous
import jax
import jax.numpy as jnp
from jax import lax
from jax.experimental import pallas as pl
from jax.experimental.pallas import tpu as pltpu
from jax.experimental.pallas import tpu_sc as plsc

N = 1024
D = 128
NC = 2
NS = 16
L = 16
NW = NC * NS
RPW = N // NW
HPW = RPW // 2
CPR = N // L


def _dist_kernel(seq_ref, q_ref, u_ref, w_ref, b_ref, d_ref, scal_ref):
    s = seq_ref[:]
    q = q_ref[:]
    u = u_ref[:]

    a = s * q
    b = s * u

    dn = (((1,), (1,)), ((), ()))
    g = lax.dot_general(a, b, dn, preferred_element_type=jnp.float32)

    ra = jnp.sum(a * a, axis=1, keepdims=True)
    rs = jnp.sum(s * s, axis=1, keepdims=True)

    ones_row = jnp.ones((1, D), dtype=jnp.float32)
    rb_t = lax.dot_general(ones_row, b * b, dn, preferred_element_type=jnp.float32)
    ones_1 = jnp.ones((1, 1), dtype=jnp.float32)
    uu_t = lax.dot_general(ones_1, u * u, dn, preferred_element_type=jnp.float32)

    d2 = jnp.maximum(ra + rb_t - 2.0 * g, 0.0)
    d = jnp.sqrt(d2)

    diag = jnp.abs(q - u) * jnp.sqrt(rs)
    row_i = lax.broadcasted_iota(jnp.int32, (N, N), 0)
    col_i = lax.broadcasted_iota(jnp.int32, (N, N), 1)
    d = jnp.where(row_i == col_i, diag, d)

    norm = jnp.mean(d)
    d_ref[:] = jnp.where(uu_t > 0.0, d, jnp.inf)

    ri = lax.broadcasted_iota(jnp.int32, (8, L), 0)
    scal = jnp.where(ri == 0, norm,
                     jnp.where(ri == 1, w_ref[0, 0],
                               jnp.where(ri == 2, b_ref[0, 0], 0.0)))
    scal_ref[:] = scal.astype(jnp.float32)


def _vgather(x, idx):
    dnums = lax.GatherDimensionNumbers(
        offset_dims=(), collapsed_slice_dims=(0,), start_index_map=(0,))
    return lax.gather(x, idx[:, None], dnums, slice_sizes=(1,),
                      mode=lax.GatherScatterMode.PROMISE_IN_BOUNDS)


def _sc_min_kernel(d_hbm, scal_hbm, out_hbm, dvm_a, dvm_b, scalvm, outvm,
                   sem_a, sem_b):
    cid = lax.axis_index("c")
    sid = lax.axis_index("s")
    wid = sid * NC + cid
    base = wid * RPW

    cp_a = pltpu.async_copy(d_hbm.at[pl.ds(base, HPW), :], dvm_a, sem_a)
    cp_b = pltpu.async_copy(d_hbm.at[pl.ds(base + HPW, HPW), :], dvm_b, sem_b)
    pltpu.sync_copy(scal_hbm.at[pl.ds(0, 3), :], scalvm)

    normv = scalvm[0, pl.ds(0, L)]
    wv = scalvm[1, pl.ds(0, L)]
    bv = scalvm[2, pl.ds(0, L)]

    lane = lax.iota(jnp.int32, L)
    inf_v = jnp.full((L,), jnp.inf, dtype=jnp.float32)
    zero = jnp.zeros((L,), dtype=jnp.float32)
    ov0, ov1 = zero, zero

    def make_rowbody(dvm):
        def rowbody(r, ov):
            mn0, mn1, mn2, mn3 = inf_v, inf_v, inf_v, inf_v
            for c in range(0, CPR, 4):
                o = c * L
                mn0 = jnp.minimum(mn0, dvm[r, pl.ds(o, L)])
                mn1 = jnp.minimum(mn1, dvm[r, pl.ds(o + L, L)])
                mn2 = jnp.minimum(mn2, dvm[r, pl.ds(o + 2 * L, L)])
                mn3 = jnp.minimum(mn3, dvm[r, pl.ds(o + 3 * L, L)])
            mn = jnp.minimum(jnp.minimum(mn0, mn1), jnp.minimum(mn2, mn3))
            for s in (8, 4, 2, 1):
                mn = jnp.minimum(mn, _vgather(mn, lane ^ s))
            return jnp.where(lane == r, mn, ov)
        return rowbody

    cp_a.wait()
    ov0 = lax.fori_loop(0, HPW, make_rowbody(dvm_a), ov0)
    cp_b.wait()
    ov1 = lax.fori_loop(0, HPW, make_rowbody(dvm_b), ov1)

    ov0 = jnp.minimum(ov0, normv)
    ov1 = jnp.minimum(ov1, normv)
    outvm[pl.ds(0, L)] = (1.0 - ov0 / normv) * wv + bv
    outvm[pl.ds(L, L)] = (1.0 - ov1 / normv) * wv + bv

    pltpu.sync_copy(outvm, out_hbm.at[pl.ds(base, RPW)])


def kernel(seq, qvs_idx, sum_idx, weight, bias):
    d, scal = pl.pallas_call(
        _dist_kernel,
        out_shape=(
            jax.ShapeDtypeStruct((N, N), jnp.float32),
            jax.ShapeDtypeStruct((8, L), jnp.float32),
        ),
    )(seq, qvs_idx, sum_idx, weight, bias.reshape(1, 1))

    mesh = plsc.VectorSubcoreMesh(core_axis_name="c", subcore_axis_name="s")
    out = pl.kernel(
        _sc_min_kernel,
        mesh=mesh,
        out_type=jax.ShapeDtypeStruct((N,), jnp.float32),
        scratch_types=[
            pltpu.VMEM((HPW, N), jnp.float32),
            pltpu.VMEM((HPW, N), jnp.float32),
            pltpu.VMEM((3, L), jnp.float32),
            pltpu.VMEM((RPW,), jnp.float32),
            pltpu.SemaphoreType.DMA,
            pltpu.SemaphoreType.DMA,
        ],
    )(d, scal)

    return out.reshape(N, 1)
